# baseline (speedup 1.0000x reference)
import functools

import jax
import jax.numpy as jnp
from jax import lax
from jax.experimental import pallas as pl
from jax.experimental.pallas import tpu as pltpu
from jax.experimental.pallas import tpu_sc as plsc

B, S, H = 4096, 200, 64
NC, NS = 2, 16
NW = NC * NS
BPW = B // NW

_mesh = plsc.VectorSubcoreMesh(core_axis_name="c", subcore_axis_name="s")


@functools.partial(
    pl.kernel,
    out_type=jax.ShapeDtypeStruct((B, H), jnp.float32),
    mesh=_mesh,
    scratch_types=[
        pltpu.VMEM((BPW, S), jnp.int32),
        pltpu.VMEM((S, H), jnp.float32),
        pltpu.VMEM((1, H), jnp.float32),
        pltpu.VMEM((BPW, H), jnp.float32),
        pltpu.SemaphoreType.DMA,
    ],
    compiler_params=pltpu.CompilerParams(use_tc_tiling_on_sc=False),
)
def _embed_sum(seqs_hbm, weight_hbm, out_hbm, idx_v, rows_v, w0_v, out_v, sem):
    wid = lax.axis_index("s") * NC + lax.axis_index("c")
    base = wid * BPW
    pltpu.sync_copy(seqs_hbm.at[pl.ds(base, BPW)], idx_v)
    pltpu.sync_copy(weight_hbm.at[pl.ds(0, 1)], w0_v)
    lane = lax.iota(jnp.int32, 16)

    def per_row(b, carry):
        cp1 = pltpu.async_copy(
            weight_hbm.at[idx_v.at[b, pl.ds(0, 128)]],
            rows_v.at[pl.ds(0, 128)], sem)
        cp2 = pltpu.async_copy(
            weight_hbm.at[idx_v.at[b, pl.ds(128, 72)]],
            rows_v.at[pl.ds(128, 72)], sem)
        cp1.wait()
        cp2.wait()

        def tok(t, accs):
            return tuple(
                accs[k] + rows_v[t, pl.ds(k * 16, 16)] for k in range(4))

        z = jnp.zeros((16,), jnp.float32)
        accs = lax.fori_loop(0, S, tok, (z, z, z, z))

        one = jnp.ones((16,), jnp.float32)
        zero = jnp.zeros((16,), jnp.float32)

        def cchunk(c, cnt):
            tv = idx_v[b, pl.ds(c * 16, 16)]
            return cnt + jnp.where(tv == 0, one, zero)

        cnt = lax.fori_loop(0, 12, cchunk, zero)
        tvt = idx_v[b, pl.ds(S - 16, 16)]
        tail = jnp.where(lane >= 8, one, zero)
        cnt = cnt + jnp.where(tvt == 0, tail, zero)
        n0 = cnt[0]
        for j in range(1, 16):
            n0 = n0 + cnt[j]
        for k in range(4):
            out_v[b, pl.ds(k * 16, 16)] = (
                accs[k] - n0 * w0_v[0, pl.ds(k * 16, 16)])
        return carry

    lax.fori_loop(0, BPW, per_row, 0)
    pltpu.sync_copy(out_v, out_hbm.at[pl.ds(base, BPW)])


def kernel(seqs, weight):
    return _embed_sum(seqs, weight)


# speedup vs baseline: 1.2450x; 1.2450x over previous
import functools

import jax
import jax.numpy as jnp
from jax import lax
from jax.experimental import pallas as pl
from jax.experimental.pallas import tpu as pltpu
from jax.experimental.pallas import tpu_sc as plsc

B, S, H = 4096, 200, 64
NC, NS = 2, 16
NW = NC * NS
BPW = B // NW
JB = BPW // 16

_mesh = plsc.VectorSubcoreMesh(core_axis_name="c", subcore_axis_name="s")


@functools.partial(
    pl.kernel,
    out_type=jax.ShapeDtypeStruct((B, H), jnp.float32),
    mesh=_mesh,
    scratch_types=[
        pltpu.VMEM((S, BPW), jnp.int32),
        pltpu.VMEM((BPW, H), jnp.float32),
        pltpu.VMEM((1, H), jnp.float32),
        pltpu.VMEM((BPW,), jnp.float32),
        pltpu.SemaphoreType.DMA,
    ],
    compiler_params=pltpu.CompilerParams(use_tc_tiling_on_sc=False),
)
def _embed_sum(seqs_t_hbm, weight_hbm, out_hbm, idx_v, acc_v, w0_v, cnt_v,
               sem):
    wid = lax.axis_index("s") * NC + lax.axis_index("c")
    base = wid * BPW
    pltpu.sync_copy(seqs_t_hbm.at[:, pl.ds(base, BPW)], idx_v)
    pltpu.sync_copy(weight_hbm.at[pl.ds(0, 1)], w0_v)

    zero = jnp.zeros((16,), jnp.float32)

    def zrow(b, carry):
        for k in range(H // 16):
            acc_v[b, pl.ds(k * 16, 16)] = zero
        return carry

    lax.fori_loop(0, BPW, zrow, 0)

    def fire(t, carry):
        pltpu.async_copy(weight_hbm.at[idx_v.at[t]], acc_v, sem, add=True)
        return carry

    lax.fori_loop(0, S, fire, 0)

    one = jnp.ones((16,), jnp.float32)

    def count(t, cnts):
        return tuple(
            cnts[j] + jnp.where(idx_v[t, pl.ds(j * 16, 16)] == 0, one, zero)
            for j in range(JB))

    cnts = lax.fori_loop(0, S, count, (zero,) * JB)

    def drain(t, carry):
        pltpu.make_async_copy(weight_hbm.at[idx_v.at[t]], acc_v, sem).wait()
        return carry

    lax.fori_loop(0, S, drain, 0)

    w0 = [w0_v[0, pl.ds(k * 16, 16)] for k in range(H // 16)]
    for j in range(JB):
        for i in range(16):
            b = j * 16 + i
            n0 = cnts[j][i]
            for k in range(H // 16):
                sl = pl.ds(k * 16, 16)
                acc_v[b, sl] = acc_v[b, sl] - n0 * w0[k]
    pltpu.sync_copy(acc_v, out_hbm.at[pl.ds(base, BPW)])


def kernel(seqs, weight):
    return _embed_sum(seqs.T, weight)
